# baseline (speedup 1.0000x reference)
import jax
import jax.numpy as jnp
from jax.experimental import pallas as pl
from jax.experimental.pallas import tpu as pltpu


def _adj_kernel(steps_per_batch, z1_ref, z2_ref, rk_ref, adj_ref, rk2_ref):
    b = pl.program_id(0) // steps_per_batch
    logits = jax.lax.dot_general(
        z1_ref[...], z2_ref[b], (((1,), (1,)), ((), ())),
        preferred_element_type=jnp.float32,
    )
    adj_ref[...] = 0.5 * jnp.tanh(0.5 * logits) + 0.5
    rk2_ref[...] = jax.nn.sigmoid(rk_ref[...])


@jax.jit
def kernel(z1, z2, rk_lgt):
    B, N, Z = z1.shape
    BM = 1024
    steps_per_batch = N // BM
    grid = (B * steps_per_batch,)

    z1f = z1.reshape(B * N, Z)

    import functools
    adj, rk2 = pl.pallas_call(
        functools.partial(_adj_kernel, steps_per_batch),
        grid=grid,
        in_specs=[
            pl.BlockSpec((BM, Z), lambda i: (i, 0)),
            pl.BlockSpec((B, N, Z), lambda i: (0, 0, 0)),
            pl.BlockSpec((1, Z), lambda i: (0, 0)),
        ],
        out_specs=[
            pl.BlockSpec((BM, N), lambda i: (i, 0)),
            pl.BlockSpec((1, Z), lambda i: (0, 0)),
        ],
        out_shape=[
            jax.ShapeDtypeStruct((B * N, N), jnp.float32),
            jax.ShapeDtypeStruct(rk_lgt.shape, jnp.float32),
        ],
        compiler_params=pltpu.CompilerParams(
            dimension_semantics=("arbitrary",),
        ),
    )(z1f, z2, rk_lgt)

    return (adj.reshape(B, N, N), z1, z2, rk2)


# speedup vs baseline: 1.0245x; 1.0245x over previous
import jax
import jax.numpy as jnp
from jax.experimental import pallas as pl
from jax.experimental.pallas import tpu as pltpu


def _adj_kernel(steps_per_batch, z1_ref, z2_ref, rk_ref, adj_ref, rk2_ref):
    b = pl.program_id(0) // steps_per_batch
    logits = jax.lax.dot_general(
        z1_ref[...], z2_ref[b], (((1,), (1,)), ((), ())),
        preferred_element_type=jnp.float32,
    )
    adj_ref[...] = 0.5 * jnp.tanh(0.5 * logits) + 0.5
    rk2_ref[...] = jax.nn.sigmoid(rk_ref[...])


@jax.jit
def kernel(z1, z2, rk_lgt):
    B, N, Z = z1.shape
    BM = 512
    steps_per_batch = N // BM
    grid = (B * steps_per_batch,)

    z1f = z1.reshape(B * N, Z)

    import functools
    adj, rk2 = pl.pallas_call(
        functools.partial(_adj_kernel, steps_per_batch),
        grid=grid,
        in_specs=[
            pl.BlockSpec((BM, Z), lambda i: (i, 0)),
            pl.BlockSpec((B, N, Z), lambda i: (0, 0, 0)),
            pl.BlockSpec((1, Z), lambda i: (0, 0)),
        ],
        out_specs=[
            pl.BlockSpec((BM, N), lambda i: (i, 0)),
            pl.BlockSpec((1, Z), lambda i: (0, 0)),
        ],
        out_shape=[
            jax.ShapeDtypeStruct((B * N, N), jnp.float32),
            jax.ShapeDtypeStruct(rk_lgt.shape, jnp.float32),
        ],
        compiler_params=pltpu.CompilerParams(
            dimension_semantics=("arbitrary",),
        ),
    )(z1f, z2, rk_lgt)

    return (adj.reshape(B, N, N), z1, z2, rk2)


# speedup vs baseline: 1.0281x; 1.0035x over previous
import jax
import jax.numpy as jnp
from jax.experimental import pallas as pl
from jax.experimental.pallas import tpu as pltpu


def _adj_kernel(steps_per_batch, z1_ref, z2_ref, rk_ref, adj_ref, rk2_ref):
    b = pl.program_id(0) // steps_per_batch
    logits = jax.lax.dot_general(
        z1_ref[...], z2_ref[b], (((1,), (1,)), ((), ())),
        preferred_element_type=jnp.float32,
    )
    adj_ref[...] = 0.5 * jnp.tanh(0.5 * logits) + 0.5
    rk2_ref[...] = jax.nn.sigmoid(rk_ref[...])


@jax.jit
def kernel(z1, z2, rk_lgt):
    B, N, Z = z1.shape
    BM = 512
    steps_per_batch = N // BM
    grid = (B * steps_per_batch,)

    z1f = z1.reshape(B * N, Z)

    import functools
    adj, rk2 = pl.pallas_call(
        functools.partial(_adj_kernel, steps_per_batch),
        grid=grid,
        in_specs=[
            pl.BlockSpec((BM, Z), lambda i: (i, 0)),
            pl.BlockSpec((B, N, Z), lambda i: (0, 0, 0)),
            pl.BlockSpec((1, Z), lambda i: (0, 0)),
        ],
        out_specs=[
            pl.BlockSpec((BM, N), lambda i: (i, 0)),
            pl.BlockSpec((1, Z), lambda i: (0, 0)),
        ],
        out_shape=[
            jax.ShapeDtypeStruct((B * N, N), jnp.float32),
            jax.ShapeDtypeStruct(rk_lgt.shape, jnp.float32),
        ],
        compiler_params=pltpu.CompilerParams(
            dimension_semantics=("parallel",),
        ),
    )(z1f, z2, rk_lgt)

    return (adj.reshape(B, N, N), z1, z2, rk2)
